# baseline (speedup 1.0000x reference)
import functools

import jax
import jax.numpy as jnp
from jax import lax
from jax.experimental import pallas as pl
from jax.experimental.pallas import tpu as pltpu
from jax.experimental.pallas import tpu_sc as plsc

_N = 10000
_E = 320000
_D = 128
_THR = 0.1
_EPS = 1e-8

_NC, _NS, _L = 2, 16, 16
_NW = _NC * _NS
_EW = _E // _NW
_EBLK = 80
_NCHUNK = _EW // _EBLK
_NGRP = _EBLK // _L

_mesh = plsc.VectorSubcoreMesh(core_axis_name="c", subcore_axis_name="s")


def _norm_body(x_ref, xn_ref):
    x = x_ref[...]
    ssq = jnp.sum(x * x, axis=1, keepdims=True)
    inv = 1.0 / jnp.maximum(jnp.sqrt(ssq), _EPS)
    xn_ref[...] = x * inv


def _normalize(x):
    blk = 2000
    return pl.pallas_call(
        _norm_body,
        grid=(_N // blk,),
        in_specs=[pl.BlockSpec((blk, _D), lambda i: (i, 0))],
        out_specs=pl.BlockSpec((blk, _D), lambda i: (i, 0)),
        out_shape=jax.ShapeDtypeStruct((_N, _D), jnp.float32),
    )(x)


_DUNROLL = 32


def _score_body(xn_hbm, row_hbm, col_hbm, att_hbm, part_hbm,
                ridx_v, cidx_v, a_v, b_v, att_v, part_v,
                sa0, sa1, sb0, sb1):
    wid = lax.axis_index("s") * _NC + lax.axis_index("c")
    ebase = wid * _EW
    sems_a = (sa0, sa1)
    sems_b = (sb0, sb1)

    def zero_body(i, c):
        part_v[pl.ds(i * _L, _L)] = jnp.zeros((_L,), jnp.float32)
        return c
    lax.fori_loop(0, _N // _L, zero_body, 0)

    iota = lax.iota(jnp.int32, _L)

    pltpu.sync_copy(row_hbm.at[pl.ds(ebase, _EW)], ridx_v)
    pltpu.sync_copy(col_hbm.at[pl.ds(ebase, _EW)], cidx_v)

    def fire(ci, b):
        sl = pl.ds(ci * _EBLK, _EBLK)
        pltpu.async_copy(xn_hbm.at[ridx_v.at[sl]], a_v.at[b], sems_a[b])
        pltpu.async_copy(xn_hbm.at[cidx_v.at[sl]], b_v.at[b], sems_b[b])

    def wait(ci, b):
        sl = pl.ds(ci * _EBLK, _EBLK)
        pltpu.make_async_copy(xn_hbm.at[ridx_v.at[sl]], a_v.at[b], sems_a[b]).wait()
        pltpu.make_async_copy(xn_hbm.at[cidx_v.at[sl]], b_v.at[b], sems_b[b]).wait()

    def compute(ci, b):
        av = a_v.at[b]
        bv = b_v.at[b]

        def grp_body(g, c2):
            lanes = g * _L + iota

            def d_body(k, carry):
                acc, dv = carry
                for _ in range(_DUNROLL):
                    ai = plsc.load_gather(av, [lanes, dv])
                    bi = plsc.load_gather(bv, [lanes, dv])
                    acc = acc + ai * bi
                    dv = jnp.bitwise_and(dv + 1, _D - 1)
                return (acc, dv)
            dot, _unused = lax.fori_loop(0, _D // _DUNROLL, d_body,
                                         (jnp.zeros((_L,), jnp.float32), iota))
            att = jnp.where(dot < _THR, 0.0, dot)
            att_v[pl.ds(ci * _EBLK + g * _L, _L)] = att
            ridx_vec = ridx_v[pl.ds(ci * _EBLK + g * _L, _L)]
            plsc.addupdate_scatter(part_v, [ridx_vec], att)
            return c2
        lax.fori_loop(0, _NGRP, grp_body, 0)

    fire(0, 0)
    fire(1, 1)

    def pair_body(p, c):
        for b in (0, 1):
            ci = 2 * p + b
            wait(ci, b)
            compute(ci, b)
            nxt = ci + 2

            @pl.when(nxt <= _NCHUNK - 1)
            def _():
                fire(nxt, b)
        return c
    lax.fori_loop(0, (_NCHUNK - 1) // 2, pair_body, 0)
    last = _NCHUNK - 1
    wait(last, last % 2)
    compute(last, last % 2)

    pltpu.sync_copy(att_v, att_hbm.at[pl.ds(ebase, _EW)])
    pltpu.sync_copy(part_v, part_hbm.at[wid])


_score_kernel = functools.partial(
    pl.kernel,
    out_type=(
        jax.ShapeDtypeStruct((_E,), jnp.float32),
        jax.ShapeDtypeStruct((_NW, _N), jnp.float32),
    ),
    mesh=_mesh,
    compiler_params=pltpu.CompilerParams(needs_layout_passes=False),
    scratch_types=[
        pltpu.VMEM((_EW,), jnp.int32),
        pltpu.VMEM((_EW,), jnp.int32),
        pltpu.VMEM((2, _EBLK, _D), jnp.float32),
        pltpu.VMEM((2, _EBLK, _D), jnp.float32),
        pltpu.VMEM((_EW,), jnp.float32),
        pltpu.VMEM((_N,), jnp.float32),
        pltpu.SemaphoreType.DMA,
        pltpu.SemaphoreType.DMA,
        pltpu.SemaphoreType.DMA,
        pltpu.SemaphoreType.DMA,
    ],
)(_score_body)


def _denom_body(part_ref, den_ref):
    s = jnp.sum(part_ref[...], axis=0, keepdims=True)
    den_ref[...] = jnp.where(s == 0.0, 1.0, s)


def _denoms(partials):
    return pl.pallas_call(
        _denom_body,
        out_shape=jax.ShapeDtypeStruct((1, _N), jnp.float32),
    )(partials)


def _final_body(att_hbm, row_hbm, den_hbm, out_hbm,
                den_v, ridx_v, att_v, out_v):
    wid = lax.axis_index("s") * _NC + lax.axis_index("c")
    ebase = wid * _EW
    pltpu.sync_copy(den_hbm, den_v)
    pltpu.sync_copy(row_hbm.at[pl.ds(ebase, _EW)], ridx_v)
    pltpu.sync_copy(att_hbm.at[pl.ds(ebase, _EW)], att_v)

    def grp_body(g, c):
        sl = pl.ds(g * _L, _L)
        ridx_vec = ridx_v[sl]
        den = plsc.load_gather(den_v, [ridx_vec])
        out_v[sl] = jnp.exp(att_v[sl] / den)
        return c
    lax.fori_loop(0, _EW // _L, grp_body, 0)
    pltpu.sync_copy(out_v, out_hbm.at[pl.ds(ebase, _EW)])


_final_kernel = functools.partial(
    pl.kernel,
    out_type=jax.ShapeDtypeStruct((_E,), jnp.float32),
    mesh=_mesh,
    compiler_params=pltpu.CompilerParams(needs_layout_passes=False),
    scratch_types=[
        pltpu.VMEM((_N,), jnp.float32),
        pltpu.VMEM((_EW,), jnp.int32),
        pltpu.VMEM((_EW,), jnp.float32),
        pltpu.VMEM((_EW,), jnp.float32),
    ],
)(_final_body)


def kernel(x, edge_index):
    row = edge_index[0]
    col = edge_index[1]
    xn = _normalize(x)
    att, partials = _score_kernel(xn, row, col)
    den = _denoms(partials).reshape(_N)
    return _final_kernel(att, row, den)


# speedup vs baseline: 1.0324x; 1.0324x over previous
import functools

import jax
import jax.numpy as jnp
from jax import lax
from jax.experimental import pallas as pl
from jax.experimental.pallas import tpu as pltpu
from jax.experimental.pallas import tpu_sc as plsc

_N = 10000
_E = 320000
_D = 128
_THR = 0.1
_EPS = 1e-8

_NC, _NS, _L = 2, 16, 16
_NW = _NC * _NS
_EW = _E // _NW
_EBLK = 80
_NCHUNK = _EW // _EBLK
_NGRP = _EBLK // _L

_mesh = plsc.VectorSubcoreMesh(core_axis_name="c", subcore_axis_name="s")


def _norm_body(x_ref, xn_ref):
    x = x_ref[...]
    ssq = jnp.sum(x * x, axis=1, keepdims=True)
    inv = 1.0 / jnp.maximum(jnp.sqrt(ssq), _EPS)
    xn_ref[...] = x * inv


def _normalize(x):
    blk = 2000
    return pl.pallas_call(
        _norm_body,
        grid=(_N // blk,),
        in_specs=[pl.BlockSpec((blk, _D), lambda i: (i, 0))],
        out_specs=pl.BlockSpec((blk, _D), lambda i: (i, 0)),
        out_shape=jax.ShapeDtypeStruct((_N, _D), jnp.float32),
    )(x)


_DUNROLL = 16


def _score_body(xn_hbm, row_hbm, col_hbm, att_hbm, part_hbm,
                ridx_v, cidx_v, a_v, b_v, att_v, part_v,
                sa0, sa1, sb0, sb1):
    wid = lax.axis_index("s") * _NC + lax.axis_index("c")
    ebase = wid * _EW
    sems_a = (sa0, sa1)
    sems_b = (sb0, sb1)

    def zero_body(i, c):
        part_v[pl.ds(i * _L, _L)] = jnp.zeros((_L,), jnp.float32)
        return c
    lax.fori_loop(0, _N // _L, zero_body, 0)

    iota = lax.iota(jnp.int32, _L)

    pltpu.sync_copy(row_hbm.at[pl.ds(ebase, _EW)], ridx_v)
    pltpu.sync_copy(col_hbm.at[pl.ds(ebase, _EW)], cidx_v)

    def fire(ci, b):
        sl = pl.ds(ci * _EBLK, _EBLK)
        pltpu.async_copy(xn_hbm.at[ridx_v.at[sl]], a_v.at[b], sems_a[b])
        pltpu.async_copy(xn_hbm.at[cidx_v.at[sl]], b_v.at[b], sems_b[b])

    def wait(ci, b):
        sl = pl.ds(ci * _EBLK, _EBLK)
        pltpu.make_async_copy(xn_hbm.at[ridx_v.at[sl]], a_v.at[b], sems_a[b]).wait()
        pltpu.make_async_copy(xn_hbm.at[cidx_v.at[sl]], b_v.at[b], sems_b[b]).wait()

    def compute(ci, b):
        av = a_v.at[b]
        bv = b_v.at[b]

        def grp_body(g, c2):
            lanes = g * _L + iota

            def d_body(k, carry):
                accs = list(carry[:4])
                dv = carry[4]
                for j in range(_DUNROLL):
                    ai = plsc.load_gather(av, [lanes, dv])
                    bi = plsc.load_gather(bv, [lanes, dv])
                    accs[j % 4] = accs[j % 4] + ai * bi
                    dv = jnp.bitwise_and(dv + 1, _D - 1)
                return (*accs, dv)
            z = jnp.zeros((_L,), jnp.float32)
            a0, a1, a2, a3, _unused = lax.fori_loop(
                0, _D // _DUNROLL, d_body, (z, z, z, z, iota))
            dot = (a0 + a1) + (a2 + a3)
            att = jnp.where(dot < _THR, 0.0, dot)
            att_v[pl.ds(ci * _EBLK + g * _L, _L)] = att
            ridx_vec = ridx_v[pl.ds(ci * _EBLK + g * _L, _L)]
            plsc.addupdate_scatter(part_v, [ridx_vec], att)
            return c2
        lax.fori_loop(0, _NGRP, grp_body, 0)

    fire(0, 0)
    fire(1, 1)

    def pair_body(p, c):
        for b in (0, 1):
            ci = 2 * p + b
            wait(ci, b)
            compute(ci, b)
            nxt = ci + 2

            @pl.when(nxt <= _NCHUNK - 1)
            def _():
                fire(nxt, b)
        return c
    lax.fori_loop(0, (_NCHUNK - 1) // 2, pair_body, 0)
    last = _NCHUNK - 1
    wait(last, last % 2)
    compute(last, last % 2)

    pltpu.sync_copy(att_v, att_hbm.at[pl.ds(ebase, _EW)])
    pltpu.sync_copy(part_v, part_hbm.at[wid])


_score_kernel = functools.partial(
    pl.kernel,
    out_type=(
        jax.ShapeDtypeStruct((_E,), jnp.float32),
        jax.ShapeDtypeStruct((_NW, _N), jnp.float32),
    ),
    mesh=_mesh,
    compiler_params=pltpu.CompilerParams(needs_layout_passes=False),
    scratch_types=[
        pltpu.VMEM((_EW,), jnp.int32),
        pltpu.VMEM((_EW,), jnp.int32),
        pltpu.VMEM((2, _EBLK, _D), jnp.float32),
        pltpu.VMEM((2, _EBLK, _D), jnp.float32),
        pltpu.VMEM((_EW,), jnp.float32),
        pltpu.VMEM((_N,), jnp.float32),
        pltpu.SemaphoreType.DMA,
        pltpu.SemaphoreType.DMA,
        pltpu.SemaphoreType.DMA,
        pltpu.SemaphoreType.DMA,
    ],
)(_score_body)


def _denom_body(part_ref, den_ref):
    s = jnp.sum(part_ref[...], axis=0, keepdims=True)
    den_ref[...] = jnp.where(s == 0.0, 1.0, s)


def _denoms(partials):
    return pl.pallas_call(
        _denom_body,
        out_shape=jax.ShapeDtypeStruct((1, _N), jnp.float32),
    )(partials)


def _final_body(att_hbm, row_hbm, den_hbm, out_hbm,
                den_v, ridx_v, att_v, out_v):
    wid = lax.axis_index("s") * _NC + lax.axis_index("c")
    ebase = wid * _EW
    pltpu.sync_copy(den_hbm, den_v)
    pltpu.sync_copy(row_hbm.at[pl.ds(ebase, _EW)], ridx_v)
    pltpu.sync_copy(att_hbm.at[pl.ds(ebase, _EW)], att_v)

    def grp_body(g, c):
        sl = pl.ds(g * _L, _L)
        ridx_vec = ridx_v[sl]
        den = plsc.load_gather(den_v, [ridx_vec])
        out_v[sl] = jnp.exp(att_v[sl] / den)
        return c
    lax.fori_loop(0, _EW // _L, grp_body, 0)
    pltpu.sync_copy(out_v, out_hbm.at[pl.ds(ebase, _EW)])


_final_kernel = functools.partial(
    pl.kernel,
    out_type=jax.ShapeDtypeStruct((_E,), jnp.float32),
    mesh=_mesh,
    compiler_params=pltpu.CompilerParams(needs_layout_passes=False),
    scratch_types=[
        pltpu.VMEM((_N,), jnp.float32),
        pltpu.VMEM((_EW,), jnp.int32),
        pltpu.VMEM((_EW,), jnp.float32),
        pltpu.VMEM((_EW,), jnp.float32),
    ],
)(_final_body)


def kernel(x, edge_index):
    row = edge_index[0]
    col = edge_index[1]
    xn = _normalize(x)
    att, partials = _score_kernel(xn, row, col)
    den = _denoms(partials).reshape(_N)
    return _final_kernel(att, row, den)


# speedup vs baseline: 1.1936x; 1.1562x over previous
import functools

import jax
import jax.numpy as jnp
from jax import lax
from jax.experimental import pallas as pl
from jax.experimental.pallas import tpu as pltpu
from jax.experimental.pallas import tpu_sc as plsc

_N = 10000
_E = 320000
_D = 128
_THR = 0.1
_EPS = 1e-8

_NC, _NS, _L = 2, 16, 16
_NW = _NC * _NS
_EW = _E // _NW
_EBLK = 80
_NCHUNK = _EW // _EBLK
_NGRP = _EBLK // _L

_mesh = plsc.VectorSubcoreMesh(core_axis_name="c", subcore_axis_name="s")


def _norm_body(x_ref, xn_ref):
    x = x_ref[...]
    ssq = jnp.sum(x * x, axis=1, keepdims=True)
    inv = 1.0 / jnp.maximum(jnp.sqrt(ssq), _EPS)
    xn_ref[...] = x * inv


def _normalize(x):
    blk = 2000
    return pl.pallas_call(
        _norm_body,
        grid=(_N // blk,),
        in_specs=[pl.BlockSpec((blk, _D), lambda i: (i, 0))],
        out_specs=pl.BlockSpec((blk, _D), lambda i: (i, 0)),
        out_shape=jax.ShapeDtypeStruct((_N, _D), jnp.float32),
    )(x)


_DUNROLL = 16


_NB = 3


def _score_body(xn_hbm, row_hbm, col_hbm, att_hbm, part_hbm,
                ridx_v, cidx_v, a_v, b_v, att_v, part_v,
                sa0, sa1, sa2, sb0, sb1, sb2):
    wid = lax.axis_index("s") * _NC + lax.axis_index("c")
    ebase = wid * _EW
    sems_a = (sa0, sa1, sa2)
    sems_b = (sb0, sb1, sb2)

    def zero_body(i, c):
        part_v[pl.ds(i * _L, _L)] = jnp.zeros((_L,), jnp.float32)
        return c
    lax.fori_loop(0, _N // _L, zero_body, 0)

    iota = lax.iota(jnp.int32, _L)

    pltpu.sync_copy(row_hbm.at[pl.ds(ebase, _EW)], ridx_v)
    pltpu.sync_copy(col_hbm.at[pl.ds(ebase, _EW)], cidx_v)

    def fire(ci, b):
        sl = pl.ds(ci * _EBLK, _EBLK)
        pltpu.async_copy(xn_hbm.at[ridx_v.at[sl]], a_v.at[b], sems_a[b])
        pltpu.async_copy(xn_hbm.at[cidx_v.at[sl]], b_v.at[b], sems_b[b])

    def wait(ci, b):
        sl = pl.ds(ci * _EBLK, _EBLK)
        pltpu.make_async_copy(xn_hbm.at[ridx_v.at[sl]], a_v.at[b], sems_a[b]).wait()
        pltpu.make_async_copy(xn_hbm.at[cidx_v.at[sl]], b_v.at[b], sems_b[b]).wait()

    def compute(ci, b):
        av = a_v.at[b]
        bv = b_v.at[b]

        def grp_body(g, c2):
            lanes = g * _L + iota

            def d_body(k, carry):
                accs = list(carry[:4])
                dv = carry[4]
                for j in range(_DUNROLL):
                    ai = plsc.load_gather(av, [lanes, dv])
                    bi = plsc.load_gather(bv, [lanes, dv])
                    accs[j % 4] = accs[j % 4] + ai * bi
                    dv = jnp.bitwise_and(dv + 1, _D - 1)
                return (*accs, dv)
            z = jnp.zeros((_L,), jnp.float32)
            a0, a1, a2, a3, _unused = lax.fori_loop(
                0, _D // _DUNROLL, d_body, (z, z, z, z, iota))
            dot = (a0 + a1) + (a2 + a3)
            att = jnp.where(dot < _THR, 0.0, dot)
            att_v[pl.ds(ci * _EBLK + g * _L, _L)] = att
            ridx_vec = ridx_v[pl.ds(ci * _EBLK + g * _L, _L)]
            plsc.addupdate_scatter(part_v, [ridx_vec], att)
            return c2
        lax.fori_loop(0, _NGRP, grp_body, 0)

    for b in range(_NB):
        fire(b, b)

    _NSTEADY = ((_NCHUNK - _NB + 1) // _NB) * _NB

    def ring_body(q, c):
        for b in range(_NB):
            ci = _NB * q + b
            wait(ci, b)
            compute(ci, b)
            nxt = ci + _NB

            @pl.when(nxt <= _NCHUNK - 1)
            def _():
                fire(nxt, b)
        return c
    lax.fori_loop(0, _NSTEADY // _NB, ring_body, 0)
    for ci in range(_NSTEADY, _NCHUNK):
        wait(ci, ci % _NB)
        compute(ci, ci % _NB)

    pltpu.sync_copy(att_v, att_hbm.at[pl.ds(ebase, _EW)])
    pltpu.sync_copy(part_v, part_hbm.at[wid])


_score_kernel = functools.partial(
    pl.kernel,
    out_type=(
        jax.ShapeDtypeStruct((_E,), jnp.float32),
        jax.ShapeDtypeStruct((_NW, _N), jnp.float32),
    ),
    mesh=_mesh,
    compiler_params=pltpu.CompilerParams(needs_layout_passes=False),
    scratch_types=[
        pltpu.VMEM((_EW,), jnp.int32),
        pltpu.VMEM((_EW,), jnp.int32),
        pltpu.VMEM((_NB, _EBLK, _D), jnp.float32),
        pltpu.VMEM((_NB, _EBLK, _D), jnp.float32),
        pltpu.VMEM((_EW,), jnp.float32),
        pltpu.VMEM((_N,), jnp.float32),
        pltpu.SemaphoreType.DMA,
        pltpu.SemaphoreType.DMA,
        pltpu.SemaphoreType.DMA,
        pltpu.SemaphoreType.DMA,
        pltpu.SemaphoreType.DMA,
        pltpu.SemaphoreType.DMA,
    ],
)(_score_body)


def _denom_body(part_ref, den_ref):
    s = jnp.sum(part_ref[...], axis=0, keepdims=True)
    den_ref[...] = jnp.where(s == 0.0, 1.0, s)


def _denoms(partials):
    return pl.pallas_call(
        _denom_body,
        out_shape=jax.ShapeDtypeStruct((1, _N), jnp.float32),
    )(partials)


def _final_body(att_hbm, row_hbm, den_hbm, out_hbm,
                den_v, ridx_v, att_v, out_v):
    wid = lax.axis_index("s") * _NC + lax.axis_index("c")
    ebase = wid * _EW
    pltpu.sync_copy(den_hbm, den_v)
    pltpu.sync_copy(row_hbm.at[pl.ds(ebase, _EW)], ridx_v)
    pltpu.sync_copy(att_hbm.at[pl.ds(ebase, _EW)], att_v)

    def grp_body(g, c):
        sl = pl.ds(g * _L, _L)
        ridx_vec = ridx_v[sl]
        den = plsc.load_gather(den_v, [ridx_vec])
        out_v[sl] = jnp.exp(att_v[sl] / den)
        return c
    lax.fori_loop(0, _EW // _L, grp_body, 0)
    pltpu.sync_copy(out_v, out_hbm.at[pl.ds(ebase, _EW)])


_final_kernel = functools.partial(
    pl.kernel,
    out_type=jax.ShapeDtypeStruct((_E,), jnp.float32),
    mesh=_mesh,
    compiler_params=pltpu.CompilerParams(needs_layout_passes=False),
    scratch_types=[
        pltpu.VMEM((_N,), jnp.float32),
        pltpu.VMEM((_EW,), jnp.int32),
        pltpu.VMEM((_EW,), jnp.float32),
        pltpu.VMEM((_EW,), jnp.float32),
    ],
)(_final_body)


def kernel(x, edge_index):
    row = edge_index[0]
    col = edge_index[1]
    xn = _normalize(x)
    att, partials = _score_kernel(xn, row, col)
    den = _denoms(partials).reshape(_N)
    return _final_kernel(att, row, den)
